# Initial kernel scaffold; baseline (speedup 1.0000x reference)
import jax
import jax.numpy as jnp
from jax.experimental import pallas as pl


def kernel(current_returns, reference_returns, w1, b1, w2, b2):
    raise NotImplementedError("write your pallas kernel here")


# speedup vs baseline: 46.1830x; 46.1830x over previous
import jax
import jax.numpy as jnp
from jax.experimental import pallas as pl
from jax.experimental.pallas import tpu as pltpu

_NB = 49
_EPS = 1e-10


def _ig_block_kernel(x_ref, ref_ref, w1_ref, b1_ref, w2_ref, b2_ref,
                     kl_ref, enc_ref):
    x = x_ref[...]
    rf = ref_ref[...]
    n = x.shape[1]

    ref_min = jnp.min(rf, axis=1, keepdims=True)
    ref_max = jnp.max(rf, axis=1, keepdims=True)
    cmin = jnp.min(x, axis=1, keepdims=True)
    cmax = jnp.max(x, axis=1, keepdims=True)
    lo = jnp.minimum(ref_min, cmin)
    hi = jnp.maximum(ref_max, cmax)
    width = (hi - lo) / _NB

    idx_c = jnp.clip(jnp.floor((x - lo) / width).astype(jnp.int32), 0, _NB - 1)
    idx_r = jnp.clip(jnp.floor((rf - lo) / width).astype(jnp.int32), 0, _NB - 1)

    inv_nw = 1.0 / (n * width)
    sp = jnp.zeros_like(width)
    sq = jnp.zeros_like(width)
    t1 = jnp.zeros_like(width)
    for j in range(_NB):
        cr = jnp.sum((idx_r == j).astype(jnp.float32), axis=1, keepdims=True)
        cc = jnp.sum((idx_c == j).astype(jnp.float32), axis=1, keepdims=True)
        p = cr * inv_nw + _EPS
        q = cc * inv_nw + _EPS
        sp = sp + p
        sq = sq + q
        t1 = t1 + p * jnp.log(p / q)
    kl = t1 / sp + jnp.log(sq / sp)

    kl_ref[...] = kl
    h = jnp.maximum(kl * w1_ref[...] + b1_ref[...], 0.0)
    enc_ref[...] = jax.lax.dot_general(
        h, w2_ref[...], (((1,), (1,)), ((), ())),
        preferred_element_type=jnp.float32) + b2_ref[...]


def kernel(current_returns, reference_returns, w1, b1, w2, b2):
    B, T = current_returns.shape
    H = w1.shape[0]
    BR = min(128, B)
    G = B // BR

    rf2 = reference_returns.reshape(1, T)
    w1r = w1.reshape(1, H)
    b1r = b1.reshape(1, H)
    b2r = b2.reshape(1, H)

    kl2, enc = pl.pallas_call(
        _ig_block_kernel,
        grid=(G,),
        in_specs=[
            pl.BlockSpec((BR, T), lambda i: (i, 0)),
            pl.BlockSpec((1, T), lambda i: (0, 0)),
            pl.BlockSpec((1, H), lambda i: (0, 0)),
            pl.BlockSpec((1, H), lambda i: (0, 0)),
            pl.BlockSpec((H, H), lambda i: (0, 0)),
            pl.BlockSpec((1, H), lambda i: (0, 0)),
        ],
        out_specs=[
            pl.BlockSpec((BR, 1), lambda i: (i, 0)),
            pl.BlockSpec((BR, H), lambda i: (i, 0)),
        ],
        out_shape=[
            jax.ShapeDtypeStruct((B, 1), jnp.float32),
            jax.ShapeDtypeStruct((B, H), jnp.float32),
        ],
        compiler_params=pltpu.CompilerParams(
            dimension_semantics=("parallel",),
            vmem_limit_bytes=56 * 1024 * 1024,
        ),
    )(current_returns, rf2, w1r, b1r, w2, b2r)
    return kl2.reshape(B), enc


# speedup vs baseline: 79.3616x; 1.7184x over previous
import jax
import jax.numpy as jnp
from jax.experimental import pallas as pl
from jax.experimental.pallas import tpu as pltpu

_NB = 49
_NG = 13
_EPS = 1e-10


def _ig_block_kernel(x_ref, ref_ref, w1_ref, b1_ref, w2_ref, b2_ref,
                     kl_ref, enc_ref):
    x = x_ref[...]
    rf = ref_ref[...]
    n = x.shape[1] * x.shape[2]

    ref_min = jnp.min(rf)
    ref_max = jnp.max(rf)
    cmin2 = jnp.min(jnp.min(x, axis=1), axis=1, keepdims=True)
    cmax2 = jnp.max(jnp.max(x, axis=1), axis=1, keepdims=True)
    lo2 = jnp.minimum(ref_min, cmin2)
    hi2 = jnp.maximum(ref_max, cmax2)
    width2 = (hi2 - lo2) / _NB
    lo = lo2[:, :, None]
    width = width2[:, :, None]

    idx_c = jnp.clip(jnp.floor((x - lo) / width).astype(jnp.int32), 0, _NB - 1)
    idx_r = jnp.clip(jnp.floor((rf - lo) / width).astype(jnp.int32), 0, _NB - 1)
    pw_c = jnp.left_shift(1, jnp.left_shift(idx_c & 3, 3))
    g_c = jnp.right_shift(idx_c, 2)
    pw_r = jnp.left_shift(1, jnp.left_shift(idx_r & 3, 3))
    g_r = jnp.right_shift(idx_r, 2)

    inv_nw = 1.0 / (n * width2)
    sp = jnp.zeros_like(width2)
    sq = jnp.zeros_like(width2)
    t1 = jnp.zeros_like(width2)
    zero = jnp.zeros_like(pw_c)
    for g in range(_NG):
        pk_c = jnp.sum(jnp.where(g_c == g, pw_c, zero), axis=1)
        pk_r = jnp.sum(jnp.where(g_r == g, pw_r, zero), axis=1)
        nf = min(4, _NB - 4 * g)
        for f in range(nf):
            cc = jnp.sum(jnp.right_shift(pk_c, 8 * f) & 255,
                         axis=1, keepdims=True).astype(jnp.float32)
            cr = jnp.sum(jnp.right_shift(pk_r, 8 * f) & 255,
                         axis=1, keepdims=True).astype(jnp.float32)
            p = cr * inv_nw + _EPS
            q = cc * inv_nw + _EPS
            sp = sp + p
            sq = sq + q
            t1 = t1 + p * jnp.log(p / q)
    kl = t1 / sp + jnp.log(sq / sp)

    kl_ref[...] = kl
    h = jnp.maximum(kl * w1_ref[...] + b1_ref[...], 0.0)
    enc_ref[...] = jax.lax.dot_general(
        h, w2_ref[...], (((1,), (1,)), ((), ())),
        preferred_element_type=jnp.float32) + b2_ref[...]


def kernel(current_returns, reference_returns, w1, b1, w2, b2):
    B, T = current_returns.shape
    H = w1.shape[0]
    C = T // 128
    BR = min(64, B)
    G = B // BR

    x3 = current_returns.reshape(B, C, 128)
    rf3 = reference_returns.reshape(1, C, 128)
    w1r = w1.reshape(1, H)
    b1r = b1.reshape(1, H)
    b2r = b2.reshape(1, H)

    kl2, enc = pl.pallas_call(
        _ig_block_kernel,
        grid=(G,),
        in_specs=[
            pl.BlockSpec((BR, C, 128), lambda i: (i, 0, 0)),
            pl.BlockSpec((1, C, 128), lambda i: (0, 0, 0)),
            pl.BlockSpec((1, H), lambda i: (0, 0)),
            pl.BlockSpec((1, H), lambda i: (0, 0)),
            pl.BlockSpec((H, H), lambda i: (0, 0)),
            pl.BlockSpec((1, H), lambda i: (0, 0)),
        ],
        out_specs=[
            pl.BlockSpec((BR, 1), lambda i: (i, 0)),
            pl.BlockSpec((BR, H), lambda i: (i, 0)),
        ],
        out_shape=[
            jax.ShapeDtypeStruct((B, 1), jnp.float32),
            jax.ShapeDtypeStruct((B, H), jnp.float32),
        ],
        compiler_params=pltpu.CompilerParams(
            dimension_semantics=("parallel",),
            vmem_limit_bytes=56 * 1024 * 1024,
        ),
    )(x3, rf3, w1r, b1r, w2, b2r)
    return kl2.reshape(B), enc


# speedup vs baseline: 116.0663x; 1.4625x over previous
import jax
import jax.numpy as jnp
from jax.experimental import pallas as pl
from jax.experimental.pallas import tpu as pltpu

_NB = 49
_NG = 13
_EPS = 1e-10


def _ig_block_kernel(x_ref, q_ref, p3_ref, w1_ref, b1_ref, w2_ref, b2_ref,
                     kl_ref, enc_ref):
    x = x_ref[...]
    qtab = q_ref[...]
    p3 = p3_ref[...]
    BR = x.shape[0]
    n = x.shape[1] * x.shape[2]
    bsz = p3.shape[0]
    nblk = p3.shape[1]

    ref_min = qtab[:, 0:1]
    ref_max = p3[bsz - 1:bsz, nblk - 1:nblk]
    cmin2 = jnp.min(jnp.min(x, axis=1), axis=1, keepdims=True)
    cmax2 = jnp.max(jnp.max(x, axis=1), axis=1, keepdims=True)
    lo2 = jnp.minimum(ref_min, cmin2)
    hi2 = jnp.maximum(ref_max, cmax2)
    width2 = (hi2 - lo2) / _NB
    lo = lo2[:, :, None]
    width = width2[:, :, None]

    idx_c = jnp.clip(jnp.floor((x - lo) / width).astype(jnp.int32), 0, _NB - 1)
    pw_c = jnp.left_shift(1, jnp.left_shift(idx_c & 3, 3))
    g_c = jnp.right_shift(idx_c, 2)

    lane64 = jax.lax.broadcasted_iota(jnp.int32, (BR, 64), 1)
    q_lanes = jnp.zeros((BR, 64), jnp.float32)
    zero = jnp.zeros_like(pw_c)
    for g in range(_NG):
        pk_c = jnp.sum(jnp.where(g_c == g, pw_c, zero), axis=1)
        nf = min(4, _NB - 4 * g)
        for f in range(nf):
            j = 4 * g + f
            cc = jnp.sum(jnp.right_shift(pk_c, 8 * f) & 255,
                         axis=1, keepdims=True).astype(jnp.float32)
            q_lanes = q_lanes + jnp.where(lane64 == j, cc, 0.0)

    ej = lo2 + (lane64 + 1).astype(jnp.float32) * width2
    qb = jnp.broadcast_to(qtab, (BR, nblk))
    pos = jnp.zeros((BR, 64), jnp.int32)
    step = nblk // 2
    while step >= 1:
        probe = jnp.take_along_axis(qb, pos + (step - 1), axis=1)
        pos = jnp.where(probe <= ej, pos + step, pos)
        step //= 2
    probe = jnp.take_along_axis(qb, jnp.minimum(pos, nblk - 1), axis=1)
    cnt_full = jnp.where(probe <= ej, pos + 1, pos)
    blk = jnp.maximum(cnt_full - 1, 0)
    inner = jnp.zeros((BR, 64), jnp.int32)
    for s in range(bsz):
        row = jnp.broadcast_to(p3[s:s + 1, :], (BR, nblk))
        gval = jnp.take_along_axis(row, blk, axis=1)
        inner = inner + jnp.where(gval <= ej, 1, 0)
    rank = (bsz * blk + inner).astype(jnp.float32)

    fz = jnp.concatenate(
        [jnp.zeros((BR, 1), jnp.float32), rank[:, :63]], axis=1)
    nall = jnp.float32(n)
    p_lanes = jnp.where(lane64 == _NB - 1, nall - fz, rank - fz)

    valid = lane64 < _NB
    inv_nw = 1.0 / (n * width2)
    p_un = jnp.where(valid, p_lanes * inv_nw + _EPS, 0.0)
    q_un = jnp.where(valid, q_lanes * inv_nw + _EPS, 0.0)
    sp = jnp.sum(p_un, axis=1, keepdims=True)
    sq = jnp.sum(q_un, axis=1, keepdims=True)
    lt = jnp.log(jnp.where(valid, p_un / q_un, 1.0))
    t1 = jnp.sum(p_un * lt, axis=1, keepdims=True)
    kl = t1 / sp + jnp.log(sq / sp)

    kl_ref[...] = kl
    h = jnp.maximum(kl * w1_ref[...] + b1_ref[...], 0.0)
    enc_ref[...] = jax.lax.dot_general(
        h, w2_ref[...], (((1,), (1,)), ((), ())),
        preferred_element_type=jnp.float32) + b2_ref[...]


def kernel(current_returns, reference_returns, w1, b1, w2, b2):
    B, T = current_returns.shape
    H = w1.shape[0]
    C = T // 128
    BR = min(64, B)
    G = B // BR
    NBLK = 128
    BSZ = T // NBLK

    x3 = current_returns.reshape(B, C, 128)
    srt = jnp.sort(reference_returns)
    qtab = srt[::BSZ].reshape(1, NBLK)
    p3 = srt.reshape(NBLK, BSZ).T
    w1r = w1.reshape(1, H)
    b1r = b1.reshape(1, H)
    b2r = b2.reshape(1, H)

    kl2, enc = pl.pallas_call(
        _ig_block_kernel,
        grid=(G,),
        in_specs=[
            pl.BlockSpec((BR, C, 128), lambda i: (i, 0, 0)),
            pl.BlockSpec((1, NBLK), lambda i: (0, 0)),
            pl.BlockSpec((BSZ, NBLK), lambda i: (0, 0)),
            pl.BlockSpec((1, H), lambda i: (0, 0)),
            pl.BlockSpec((1, H), lambda i: (0, 0)),
            pl.BlockSpec((H, H), lambda i: (0, 0)),
            pl.BlockSpec((1, H), lambda i: (0, 0)),
        ],
        out_specs=[
            pl.BlockSpec((BR, 1), lambda i: (i, 0)),
            pl.BlockSpec((BR, H), lambda i: (i, 0)),
        ],
        out_shape=[
            jax.ShapeDtypeStruct((B, 1), jnp.float32),
            jax.ShapeDtypeStruct((B, H), jnp.float32),
        ],
        compiler_params=pltpu.CompilerParams(
            dimension_semantics=("parallel",),
            vmem_limit_bytes=56 * 1024 * 1024,
        ),
    )(x3, qtab, p3, w1r, b1r, w2, b2r)
    return kl2.reshape(B), enc


# speedup vs baseline: 166.9209x; 1.4382x over previous
import jax
import jax.numpy as jnp
from jax.experimental import pallas as pl
from jax.experimental.pallas import tpu as pltpu

_NB = 49
_NG = 13
_EPS = 1e-10


def _ig_block_kernel(x_ref, q_ref, p3_ref, w1_ref, b1_ref, w2_ref, b2_ref,
                     kl_ref, enc_ref):
    x = x_ref[...]
    qtab = q_ref[...]
    p3 = p3_ref[...]
    BR = x.shape[0]
    n = x.shape[1]
    nchunk = n // 128
    bsz = p3.shape[0]
    nblk = p3.shape[1]

    ref_min = qtab[:, 0:1]
    ref_max = p3[bsz - 1:bsz, nblk - 1:nblk]
    cmin2 = jnp.min(x, axis=1, keepdims=True)
    cmax2 = jnp.max(x, axis=1, keepdims=True)
    lo2 = jnp.minimum(ref_min, cmin2)
    hi2 = jnp.maximum(ref_max, cmax2)
    width2 = (hi2 - lo2) / _NB

    idx_c = jnp.clip(jnp.floor((x - lo2) / width2).astype(jnp.int32), 0, _NB - 1)
    pw_c = jnp.left_shift(1, jnp.left_shift(idx_c & 3, 3))
    g_c = jnp.right_shift(idx_c, 2)

    lane64 = jax.lax.broadcasted_iota(jnp.int32, (BR, 64), 1)
    q_lanes = jnp.zeros((BR, 64), jnp.float32)
    zero = jnp.zeros_like(pw_c)
    for g in range(_NG):
        sel = jnp.where(g_c == g, pw_c, zero)
        parts = [sel[:, i * 128:(i + 1) * 128] for i in range(nchunk)]
        while len(parts) > 1:
            parts = [parts[i] + parts[i + 1] for i in range(0, len(parts), 2)]
        pk_c = parts[0]
        ue = pk_c & 0x00FF00FF
        uo = jnp.right_shift(pk_c, 8) & 0x00FF00FF
        se = jnp.sum(ue, axis=1, keepdims=True)
        so = jnp.sum(uo, axis=1, keepdims=True)
        cnt4 = [se & 0xFFFF, so & 0xFFFF,
                jnp.right_shift(se, 16), jnp.right_shift(so, 16)]
        nf = min(4, _NB - 4 * g)
        for f in range(nf):
            j = 4 * g + f
            q_lanes = q_lanes + jnp.where(lane64 == j,
                                          cnt4[f].astype(jnp.float32), 0.0)

    ej = lo2 + (lane64 + 1).astype(jnp.float32) * width2
    qb = jnp.broadcast_to(qtab, (BR, nblk))
    pos = jnp.zeros((BR, 64), jnp.int32)
    step = nblk // 2
    while step >= 1:
        probe = jnp.take_along_axis(qb, pos + (step - 1), axis=1)
        pos = jnp.where(probe <= ej, pos + step, pos)
        step //= 2
    probe = jnp.take_along_axis(qb, jnp.minimum(pos, nblk - 1), axis=1)
    cnt_full = jnp.where(probe <= ej, pos + 1, pos)
    blk = jnp.maximum(cnt_full - 1, 0)
    inner = jnp.zeros((BR, 64), jnp.int32)
    for s in range(bsz):
        row = jnp.broadcast_to(p3[s:s + 1, :], (BR, nblk))
        gval = jnp.take_along_axis(row, blk, axis=1)
        inner = inner + jnp.where(gval <= ej, 1, 0)
    rank = (bsz * blk + inner).astype(jnp.float32)

    fz = jnp.concatenate(
        [jnp.zeros((BR, 1), jnp.float32), rank[:, :63]], axis=1)
    nall = jnp.float32(n)
    p_lanes = jnp.where(lane64 == _NB - 1, nall - fz, rank - fz)

    valid = lane64 < _NB
    inv_nw = 1.0 / (n * width2)
    p_un = jnp.where(valid, p_lanes * inv_nw + _EPS, 0.0)
    q_un = jnp.where(valid, q_lanes * inv_nw + _EPS, 0.0)
    sp = jnp.sum(p_un, axis=1, keepdims=True)
    sq = jnp.sum(q_un, axis=1, keepdims=True)
    lt = jnp.log(jnp.where(valid, p_un / q_un, 1.0))
    t1 = jnp.sum(p_un * lt, axis=1, keepdims=True)
    kl = t1 / sp + jnp.log(sq / sp)

    kl_ref[...] = kl
    h = jnp.maximum(kl * w1_ref[...] + b1_ref[...], 0.0)
    enc_ref[...] = jax.lax.dot_general(
        h, w2_ref[...], (((1,), (1,)), ((), ())),
        preferred_element_type=jnp.float32) + b2_ref[...]


def kernel(current_returns, reference_returns, w1, b1, w2, b2):
    B, T = current_returns.shape
    H = w1.shape[0]
    BR = min(64, B)
    G = B // BR
    NBLK = 128
    BSZ = T // NBLK

    srt = jnp.sort(reference_returns)
    qtab = srt[::BSZ].reshape(1, NBLK)
    p3 = srt.reshape(NBLK, BSZ).T
    w1r = w1.reshape(1, H)
    b1r = b1.reshape(1, H)
    b2r = b2.reshape(1, H)

    kl2, enc = pl.pallas_call(
        _ig_block_kernel,
        grid=(G,),
        in_specs=[
            pl.BlockSpec((BR, T), lambda i: (i, 0)),
            pl.BlockSpec((1, NBLK), lambda i: (0, 0)),
            pl.BlockSpec((BSZ, NBLK), lambda i: (0, 0)),
            pl.BlockSpec((1, H), lambda i: (0, 0)),
            pl.BlockSpec((1, H), lambda i: (0, 0)),
            pl.BlockSpec((H, H), lambda i: (0, 0)),
            pl.BlockSpec((1, H), lambda i: (0, 0)),
        ],
        out_specs=[
            pl.BlockSpec((BR, 1), lambda i: (i, 0)),
            pl.BlockSpec((BR, H), lambda i: (i, 0)),
        ],
        out_shape=[
            jax.ShapeDtypeStruct((B, 1), jnp.float32),
            jax.ShapeDtypeStruct((B, H), jnp.float32),
        ],
        compiler_params=pltpu.CompilerParams(
            dimension_semantics=("parallel",),
            vmem_limit_bytes=56 * 1024 * 1024,
        ),
    )(current_returns, qtab, p3, w1r, b1r, w2, b2r)
    return kl2.reshape(B), enc
